# baseline (speedup 1.0000x reference)
import jax
import jax.numpy as jnp
from jax import lax
from jax.experimental import pallas as pl
from jax.experimental.pallas import tpu as pltpu
from jax.experimental.pallas import tpu_sc as plsc

NC = 2
NS = 16
L = 16
NW = NC * NS

B = 16384
D = 64
H = 256
BPW = B // NW

BLK = 2048
GRID = B // BLK


def _mix_body(uidx_hbm, iidx_hbm, utab_hbm, itab_hbm, out_hbm,
              uidx_v, iidx_v, urows_v, irows_v, usem, isem):
    wid = lax.axis_index("s") * NC + lax.axis_index("c")
    base = wid * BPW
    pltpu.sync_copy(uidx_hbm.at[pl.ds(base, BPW)], uidx_v)
    pltpu.sync_copy(iidx_hbm.at[pl.ds(base, BPW)], iidx_v)
    cu = pltpu.async_copy(utab_hbm.at[uidx_v], urows_v, usem)
    ci = pltpu.async_copy(itab_hbm.at[iidx_v], irows_v, isem)
    cu.wait()
    ci.wait()

    def row(i, carry):
        for j in range(D // L):
            sl = (i, pl.ds(j * L, L))
            urows_v[sl] = urows_v[sl] * irows_v[sl]
        return carry

    lax.fori_loop(0, BPW, row, 0)
    pltpu.sync_copy(urows_v, out_hbm.at[pl.ds(base, BPW)])


_mix_call = pl.kernel(
    _mix_body,
    mesh=plsc.VectorSubcoreMesh(core_axis_name="c", subcore_axis_name="s"),
    compiler_params=pltpu.CompilerParams(use_tc_tiling_on_sc=False),
    out_type=jax.ShapeDtypeStruct((B, D), jnp.float32),
    scratch_types=[
        pltpu.VMEM((BPW,), jnp.int32),
        pltpu.VMEM((BPW,), jnp.int32),
        pltpu.VMEM((BPW, D), jnp.float32),
        pltpu.VMEM((BPW, D), jnp.float32),
        pltpu.SemaphoreType.DMA,
        pltpu.SemaphoreType.DMA,
    ],
)


def _mlp_body(mix_ref, w1_ref, b1_ref, w2_ref, b2_ref, out_ref):
    h = jnp.dot(mix_ref[...], w1_ref[...], preferred_element_type=jnp.float32)
    h = jnp.maximum(h + b1_ref[...], 0.0)
    z = jnp.dot(h, w2_ref[...], preferred_element_type=jnp.float32)
    out_ref[...] = jax.nn.sigmoid(z + b2_ref[...])


def _mlp(mix, W1, b1, W2, b2):
    return pl.pallas_call(
        _mlp_body,
        grid=(GRID,),
        in_specs=[
            pl.BlockSpec((BLK, D), lambda i: (i, 0)),
            pl.BlockSpec((D, H), lambda i: (0, 0)),
            pl.BlockSpec((1, H), lambda i: (0, 0)),
            pl.BlockSpec((H, 1), lambda i: (0, 0)),
            pl.BlockSpec((1, 1), lambda i: (0, 0)),
        ],
        out_specs=pl.BlockSpec((BLK, 1), lambda i: (i, 0)),
        out_shape=jax.ShapeDtypeStruct((B, 1), jnp.float32),
    )(mix, W1, b1.reshape(1, H), W2, b2.reshape(1, 1))


def kernel(user, item, user_table, item_table, W1, b1, W2, b2):
    user = user.astype(jnp.int32)
    item = item.astype(jnp.int32)
    mix = _mix_call(user, item, user_table, item_table)
    out = _mlp(mix, W1, b1, W2, b2)
    return out.reshape(-1)


# speedup vs baseline: 1.5631x; 1.5631x over previous
import jax
import jax.numpy as jnp
from jax import lax
from jax.experimental import pallas as pl
from jax.experimental.pallas import tpu as pltpu
from jax.experimental.pallas import tpu_sc as plsc

NC = 2
NS = 16
L = 16
NW = NC * NS

B = 16384
D = 64
H = 256
BPW = B // NW

CH = 256
NCHUNK = BPW // CH

BLK = 2048
GRID = B // BLK


def _mix_body(uidx_hbm, iidx_hbm, utab_hbm, itab_hbm, out_hbm,
              uidx_v, iidx_v, urows_v, irows_v, usem, isem):
    wid = lax.axis_index("s") * NC + lax.axis_index("c")
    base = wid * BPW
    pltpu.sync_copy(uidx_hbm.at[pl.ds(base, BPW)], uidx_v)
    pltpu.sync_copy(iidx_hbm.at[pl.ds(base, BPW)], iidx_v)

    def chunk(c, carry0):
        cbase = c * CH

        def issue16(k, carry):
            uvec = uidx_v[pl.ds(cbase + k * L, L)]
            ivec = iidx_v[pl.ds(cbase + k * L, L)]
            for j in range(L):
                u = uvec[j]
                pltpu.async_copy(utab_hbm.at[pl.ds(u, 1)],
                                 urows_v.at[pl.ds(k * L + j, 1)], usem)
                it = ivec[j]
                pltpu.async_copy(itab_hbm.at[pl.ds(it, 1)],
                                 irows_v.at[pl.ds(k * L + j, 1)], isem)
            return carry

        lax.fori_loop(0, CH // L, issue16, 0)
        pltpu.make_async_copy(utab_hbm.at[pl.ds(0, CH)], urows_v, usem).wait()
        pltpu.make_async_copy(itab_hbm.at[pl.ds(0, CH)], irows_v, isem).wait()

        def row(i, carry):
            for j in range(D // L):
                sl = (i, pl.ds(j * L, L))
                urows_v[sl] = urows_v[sl] * irows_v[sl]
            pltpu.async_copy(urows_v.at[pl.ds(i, 1)],
                             out_hbm.at[pl.ds(base + cbase + i, 1)], isem)
            return carry

        lax.fori_loop(0, CH, row, 0)
        pltpu.make_async_copy(out_hbm.at[pl.ds(0, CH)], irows_v, isem).wait()
        return carry0

    lax.fori_loop(0, NCHUNK, chunk, 0)


_mix_call = pl.kernel(
    _mix_body,
    mesh=plsc.VectorSubcoreMesh(core_axis_name="c", subcore_axis_name="s"),
    out_type=jax.ShapeDtypeStruct((B, D), jnp.float32),
    scratch_types=[
        pltpu.VMEM((BPW,), jnp.int32),
        pltpu.VMEM((BPW,), jnp.int32),
        pltpu.VMEM((CH, D), jnp.float32),
        pltpu.VMEM((CH, D), jnp.float32),
        pltpu.SemaphoreType.DMA,
        pltpu.SemaphoreType.DMA,
    ],
)


def _mlp_body(mix_ref, w1_ref, b1_ref, w2_ref, b2_ref, out_ref):
    h = jnp.dot(mix_ref[...], w1_ref[...], preferred_element_type=jnp.float32)
    h = jnp.maximum(h + b1_ref[...], 0.0)
    z = jnp.dot(h, w2_ref[...], preferred_element_type=jnp.float32)
    out_ref[...] = jax.nn.sigmoid(z + b2_ref[...])


def _mlp(mix, W1, b1, W2, b2):
    return pl.pallas_call(
        _mlp_body,
        grid=(GRID,),
        in_specs=[
            pl.BlockSpec((BLK, D), lambda i: (i, 0)),
            pl.BlockSpec((D, H), lambda i: (0, 0)),
            pl.BlockSpec((1, H), lambda i: (0, 0)),
            pl.BlockSpec((H, 1), lambda i: (0, 0)),
            pl.BlockSpec((1, 1), lambda i: (0, 0)),
        ],
        out_specs=pl.BlockSpec((BLK, 1), lambda i: (i, 0)),
        out_shape=jax.ShapeDtypeStruct((B, 1), jnp.float32),
    )(mix, W1, b1.reshape(1, H), W2, b2.reshape(1, 1))


def kernel(user, item, user_table, item_table, W1, b1, W2, b2):
    user = user.astype(jnp.int32)
    item = item.astype(jnp.int32)
    mix = _mix_call(user, item, user_table, item_table)
    out = _mlp(mix, W1, b1, W2, b2)
    return out.reshape(-1)
